# Initial kernel scaffold; baseline (speedup 1.0000x reference)
import jax
import jax.numpy as jnp
from jax.experimental import pallas as pl


def kernel(x, edge_index, Wq, bq, Wk, bk, Wv, bv, Wo, bo, W1, b1, W2, b2, g1, beta1, g2, beta2):
    raise NotImplementedError("write your pallas kernel here")


# speedup vs baseline: 45.6680x; 45.6680x over previous
import functools

import jax
import jax.numpy as jnp
from jax import lax
from jax.experimental import pallas as pl
from jax.experimental.pallas import tpu as pltpu
from jax.experimental.pallas import tpu_sc as plsc

N = 10000
E = 320000
D = 128
H = 8
OC = D // H
FF = 512
SCALE = 1.0 / (OC ** 0.5)

NC, NS, L = 2, 16, 16
NW = NC * NS
EPW = E // NW
B = 80
NCHUNK = EPW // B
DEN0 = N
TROWS = 11264
SLAB = TROWS // NS


def _qkv_body(x_ref, w_ref, b_ref, q_ref, k_ref, v_ref):
    y = jnp.dot(x_ref[...], w_ref[...], preferred_element_type=jnp.float32)
    y = y + b_ref[...]
    q_ref[...] = y[:, :D] * SCALE
    k_ref[...] = y[:, D:2 * D]
    v_ref[...] = y[:, 2 * D:]


def _qkv(x, wqkv_t, bqkv):
    bm = 1000
    return pl.pallas_call(
        _qkv_body,
        grid=(N // bm,),
        in_specs=[
            pl.BlockSpec((bm, D), lambda i: (i, 0)),
            pl.BlockSpec((D, 3 * D), lambda i: (0, 0)),
            pl.BlockSpec((1, 3 * D), lambda i: (0, 0)),
        ],
        out_specs=[
            pl.BlockSpec((bm, D), lambda i: (i, 0)),
            pl.BlockSpec((bm, D), lambda i: (i, 0)),
            pl.BlockSpec((bm, D), lambda i: (i, 0)),
        ],
        out_shape=[
            jax.ShapeDtypeStruct((N, D), jnp.float32),
            jax.ShapeDtypeStruct((N, D), jnp.float32),
            jax.ShapeDtypeStruct((N, D), jnp.float32),
        ],
    )(x, wqkv_t, bqkv)


def _edge_body(qtab, ktab, vtab, srcv, dstv, out, sidx, didx, denid,
               qr, kr, vr, acc, sem):
    c = lax.axis_index("c")
    s = lax.axis_index("s")
    wid = c * NS + s

    zero = jnp.zeros((L,), jnp.float32)

    def zrow(i, _):
        for j in range(D // L):
            vr[i, pl.ds(j * L, L)] = zero
        return 0

    lax.fori_loop(0, B, zrow, 0)

    base = s * SLAB

    def zcopy(k, _):
        pltpu.sync_copy(vr, acc.at[pl.ds(base + k * B, B)])
        return 0

    lax.fori_loop(0, SLAB // B, zcopy, 0)
    rem = SLAB % B
    pltpu.sync_copy(vr.at[pl.ds(0, rem)],
                    acc.at[pl.ds(base + SLAB - rem, rem)])

    plsc.subcore_barrier()

    e0 = wid * EPW
    xor8 = lax.iota(jnp.int32, L) ^ 8

    def chunk(g, _):
        eb = e0 + g * B
        pltpu.sync_copy(srcv.at[pl.ds(eb, B)], sidx)
        pltpu.sync_copy(dstv.at[pl.ds(eb, B)], didx)
        cp1 = pltpu.async_copy(qtab.at[didx], qr, sem)
        cp2 = pltpu.async_copy(ktab.at[sidx], kr, sem)
        cp3 = pltpu.async_copy(vtab.at[sidx], vr, sem)
        for gg in range(B // L):
            denid[pl.ds(gg * L, L)] = (didx[pl.ds(gg * L, L)] >> 3) + DEN0
        cp1.wait()
        cp2.wait()
        cp3.wait()

        def edge(e, _):
            a = qr[e, pl.ds(0, L)] * kr[e, pl.ds(0, L)]
            for j in range(1, D // L):
                a = a + qr[e, pl.ds(j * L, L)] * kr[e, pl.ds(j * L, L)]
            a = a + jnp.take_along_axis(a, xor8, axis=0,
                                        mode='promise_in_bounds')
            ex = jnp.exp(a)
            for j in range(D // L):
                vr[e, pl.ds(j * L, L)] = vr[e, pl.ds(j * L, L)] * ex
            grp = (e // L) * L
            dmv = didx[pl.ds(grp, L)] & 7
            bm = jnp.take_along_axis(dmv, jnp.full((L,), e - grp, jnp.int32),
                                     axis=0, mode='promise_in_bounds')
            bmf = bm.astype(jnp.float32)
            for j in range(D // L):
                m = jnp.maximum(1.0 - jnp.abs(bmf - float(j)), 0.0)
                kr[e, pl.ds(j * L, L)] = ex * m
            return 0

        lax.fori_loop(0, B, edge, 0)
        pltpu.sync_copy(vr, acc.at[didx], add=True)
        pltpu.sync_copy(kr, acc.at[denid], add=True)
        return 0

    lax.fori_loop(0, NCHUNK, chunk, 0)
    plsc.subcore_barrier()
    pltpu.sync_copy(acc.at[pl.ds(base, SLAB)], out.at[c, pl.ds(base, SLAB)])


def _edges(qtab, ktab, vtab, src, dst):
    mesh = plsc.VectorSubcoreMesh(core_axis_name="c", subcore_axis_name="s")
    f = pl.kernel(
        _edge_body,
        out_type=jax.ShapeDtypeStruct((NC, TROWS, D), jnp.float32),
        mesh=mesh,
        scratch_types=[
            pltpu.VMEM((B,), jnp.int32),
            pltpu.VMEM((B,), jnp.int32),
            pltpu.VMEM((B,), jnp.int32),
            pltpu.VMEM((B, D), jnp.float32),
            pltpu.VMEM((B, D), jnp.float32),
            pltpu.VMEM((B, D), jnp.float32),
            pltpu.VMEM_SHARED((TROWS, D), jnp.float32),
            pltpu.SemaphoreType.DMA,
        ],
    )
    return f(qtab, ktab, vtab, src, dst)


def _ln(x, g, b):
    mu = jnp.mean(x, axis=-1, keepdims=True)
    var = jnp.mean((x - mu) ** 2, axis=-1, keepdims=True)
    return (x - mu) / jnp.sqrt(var + 1e-5) * g + b


def _tail_body(x_ref, a0_ref, a1_ref, d0_ref, d1_ref, wo_ref, bo_ref,
               w1_ref, b1_ref, w2_ref, b2_ref, g1_ref, be1_ref, g2_ref,
               be2_ref, o_ref):
    den = d0_ref[...] + d1_ref[...]
    rep = (lax.broadcasted_iota(jnp.int32, (H, D), 1) % H
           == lax.broadcasted_iota(jnp.int32, (H, D), 0)).astype(jnp.float32)
    denr = jnp.dot(den, rep, preferred_element_type=jnp.float32)
    agg = (a0_ref[...] + a1_ref[...]) / (denr + 1e-16)
    att = jnp.dot(agg, wo_ref[...], preferred_element_type=jnp.float32) + bo_ref[...]
    x1 = x_ref[...] + att
    xn = _ln(x1, g1_ref[...], be1_ref[...])
    hh = jax.nn.gelu(jnp.dot(xn, w1_ref[...], preferred_element_type=jnp.float32)
                     + b1_ref[...])
    ff = jnp.dot(hh, w2_ref[...], preferred_element_type=jnp.float32) + b2_ref[...]
    o_ref[...] = _ln(xn + ff, g2_ref[...], be2_ref[...])


def _tail(x, a0, a1, d0, d1, wo_t, bo, w1_t, b1, w2_t, b2, g1, be1, g2, be2):
    bm = 1000
    row = lambda w: pl.BlockSpec((bm, w), lambda i: (i, 0))
    full = lambda a, b: pl.BlockSpec((a, b), lambda i: (0, 0))
    return pl.pallas_call(
        _tail_body,
        grid=(N // bm,),
        in_specs=[
            row(D), row(D), row(D), row(H), row(H),
            full(D, D), full(1, D),
            full(D, FF), full(1, FF),
            full(FF, D), full(1, D),
            full(1, D), full(1, D), full(1, D), full(1, D),
        ],
        out_specs=pl.BlockSpec((bm, D), lambda i: (i, 0)),
        out_shape=jax.ShapeDtypeStruct((N, D), jnp.float32),
    )(x, a0, a1, d0, d1, wo_t, bo, w1_t, b1, w2_t, b2, g1, be1, g2, be2)


def kernel(x, edge_index, Wq, bq, Wk, bk, Wv, bv, Wo, bo, W1, b1, W2, b2,
           g1, beta1, g2, beta2):
    src = edge_index[0].astype(jnp.int32)
    dst = edge_index[1].astype(jnp.int32)
    ar = jnp.arange(D)
    perm = (ar % H) * OC + ar // H
    wqkv_t = jnp.concatenate(
        [Wq.T[:, perm], Wk.T[:, perm], Wv.T[:, perm]], axis=1)
    bqkv = jnp.concatenate([bq[perm], bk[perm], bv[perm]])[None, :]
    qtab, ktab, vtab = _qkv(x, wqkv_t, bqkv)
    acc_p = _edges(qtab, ktab, vtab, src, dst)
    a0 = acc_p[0, :N, :]
    a1 = acc_p[1, :N, :]
    den = acc_p[:, DEN0:DEN0 + N // 8, :].reshape(NC, N // 8, 8, L)
    den = den[..., :H].reshape(NC, N, H)
    d0 = den[0]
    d1 = den[1]
    return _tail(x, a0, a1, d0, d1, Wo.T[perm], bo[None, :], W1.T, b1[None, :],
                 W2.T, b2[None, :], g1[None, :], beta1[None, :],
                 g2[None, :], beta2[None, :])


# speedup vs baseline: 48.1980x; 1.0554x over previous
import functools

import jax
import jax.numpy as jnp
from jax import lax
from jax.experimental import pallas as pl
from jax.experimental.pallas import tpu as pltpu
from jax.experimental.pallas import tpu_sc as plsc

N = 10000
E = 320000
D = 128
H = 8
OC = D // H
FF = 512
SCALE = 1.0 / (OC ** 0.5)

NC, NS, L = 2, 16, 16
NW = NC * NS
EPW = E // NW
B = 80
NCHUNK = EPW // B
DEN0 = N
TROWS = 11264
SLAB = TROWS // NS


def _qkv_body(x_ref, w_ref, b_ref, q_ref, k_ref, v_ref):
    y = jnp.dot(x_ref[...], w_ref[...], preferred_element_type=jnp.float32)
    y = y + b_ref[...]
    q_ref[...] = y[:, :D] * SCALE
    k_ref[...] = y[:, D:2 * D]
    v_ref[...] = y[:, 2 * D:]


def _qkv(x, wqkv_t, bqkv):
    bm = 1000
    return pl.pallas_call(
        _qkv_body,
        grid=(N // bm,),
        in_specs=[
            pl.BlockSpec((bm, D), lambda i: (i, 0)),
            pl.BlockSpec((D, 3 * D), lambda i: (0, 0)),
            pl.BlockSpec((1, 3 * D), lambda i: (0, 0)),
        ],
        out_specs=[
            pl.BlockSpec((bm, D), lambda i: (i, 0)),
            pl.BlockSpec((bm, D), lambda i: (i, 0)),
            pl.BlockSpec((bm, D), lambda i: (i, 0)),
        ],
        out_shape=[
            jax.ShapeDtypeStruct((N, D), jnp.float32),
            jax.ShapeDtypeStruct((N, D), jnp.float32),
            jax.ShapeDtypeStruct((N, D), jnp.float32),
        ],
    )(x, wqkv_t, bqkv)


def _edge_body(qtab, ktab, vtab, srcv, dstv, out, sidx, didx, denid,
               qr, kr, vr, acc, sem):
    c = lax.axis_index("c")
    s = lax.axis_index("s")
    wid = c * NS + s

    zero = jnp.zeros((L,), jnp.float32)

    def zrow(i, _):
        for j in range(D // L):
            vr[i, pl.ds(j * L, L)] = zero
        return 0

    lax.fori_loop(0, B, zrow, 0)

    base = s * SLAB

    def zcopy(k, _):
        pltpu.sync_copy(vr, acc.at[pl.ds(base + k * B, B)])
        return 0

    lax.fori_loop(0, SLAB // B, zcopy, 0)
    rem = SLAB % B
    pltpu.sync_copy(vr.at[pl.ds(0, rem)],
                    acc.at[pl.ds(base + SLAB - rem, rem)])

    plsc.subcore_barrier()

    e0 = wid * EPW
    iota = lax.iota(jnp.int32, L)
    xor8 = iota ^ 8
    self_v = jnp.maximum(1.0 - iota.astype(jnp.float32), 0.0)

    def chunk(g, _):
        eb = e0 + g * B
        pltpu.sync_copy(srcv.at[pl.ds(eb, B)], sidx)
        pltpu.sync_copy(dstv.at[pl.ds(eb, B)], didx)
        cp1 = pltpu.async_copy(qtab.at[didx], qr, sem)
        cp2 = pltpu.async_copy(ktab.at[sidx], kr, sem)
        cp3 = pltpu.async_copy(vtab.at[sidx], vr, sem)
        for gg in range(B // L):
            denid[pl.ds(gg * L, L)] = (didx[pl.ds(gg * L, L)] >> 3) + DEN0
        cp1.wait()
        cp2.wait()
        cp3.wait()

        def edge4(t, _):
            grp = (t // 4) * L
            dmv = didx[pl.ds(grp, L)] & 7
            for u in range(4):
                e = t * 4 + u
                a = qr[e, pl.ds(0, L)] * kr[e, pl.ds(0, L)]
                for j in range(1, D // L):
                    a = a + qr[e, pl.ds(j * L, L)] * kr[e, pl.ds(j * L, L)]
                a = a + jnp.take_along_axis(a, xor8, axis=0,
                                            mode='promise_in_bounds')
                ex = jnp.exp(a)
                for j in range(D // L):
                    vr[e, pl.ds(j * L, L)] = vr[e, pl.ds(j * L, L)] * ex
                bm = jnp.take_along_axis(
                    dmv, jnp.full((L,), e - grp, jnp.int32), axis=0,
                    mode='promise_in_bounds')
                oh = jnp.take_along_axis(self_v, iota ^ bm, axis=0,
                                         mode='promise_in_bounds')
                for j in range(D // L):
                    m = jnp.take_along_axis(oh, jnp.full((L,), j, jnp.int32),
                                            axis=0,
                                            mode='promise_in_bounds')
                    kr[e, pl.ds(j * L, L)] = ex * m
            return 0

        lax.fori_loop(0, B // 4, edge4, 0)
        pltpu.sync_copy(vr, acc.at[didx], add=True)
        pltpu.sync_copy(kr, acc.at[denid], add=True)
        return 0

    lax.fori_loop(0, NCHUNK, chunk, 0)
    plsc.subcore_barrier()
    pltpu.sync_copy(acc.at[pl.ds(base, SLAB)], out.at[c, pl.ds(base, SLAB)])


def _edges(qtab, ktab, vtab, src, dst):
    mesh = plsc.VectorSubcoreMesh(core_axis_name="c", subcore_axis_name="s")
    f = pl.kernel(
        _edge_body,
        out_type=jax.ShapeDtypeStruct((NC, TROWS, D), jnp.float32),
        mesh=mesh,
        scratch_types=[
            pltpu.VMEM((B,), jnp.int32),
            pltpu.VMEM((B,), jnp.int32),
            pltpu.VMEM((B,), jnp.int32),
            pltpu.VMEM((B, D), jnp.float32),
            pltpu.VMEM((B, D), jnp.float32),
            pltpu.VMEM((B, D), jnp.float32),
            pltpu.VMEM_SHARED((TROWS, D), jnp.float32),
            pltpu.SemaphoreType.DMA,
        ],
    )
    return f(qtab, ktab, vtab, src, dst)


def _ln(x, g, b):
    mu = jnp.mean(x, axis=-1, keepdims=True)
    var = jnp.mean((x - mu) ** 2, axis=-1, keepdims=True)
    return (x - mu) / jnp.sqrt(var + 1e-5) * g + b


def _tail_body(x_ref, a0_ref, a1_ref, d0_ref, d1_ref, wo_ref, bo_ref,
               w1_ref, b1_ref, w2_ref, b2_ref, g1_ref, be1_ref, g2_ref,
               be2_ref, o_ref):
    den = d0_ref[...] + d1_ref[...]
    rep = (lax.broadcasted_iota(jnp.int32, (H, D), 1) % H
           == lax.broadcasted_iota(jnp.int32, (H, D), 0)).astype(jnp.float32)
    denr = jnp.dot(den, rep, preferred_element_type=jnp.float32)
    agg = (a0_ref[...] + a1_ref[...]) / (denr + 1e-16)
    att = jnp.dot(agg, wo_ref[...], preferred_element_type=jnp.float32) + bo_ref[...]
    x1 = x_ref[...] + att
    xn = _ln(x1, g1_ref[...], be1_ref[...])
    hh = jax.nn.gelu(jnp.dot(xn, w1_ref[...], preferred_element_type=jnp.float32)
                     + b1_ref[...])
    ff = jnp.dot(hh, w2_ref[...], preferred_element_type=jnp.float32) + b2_ref[...]
    o_ref[...] = _ln(xn + ff, g2_ref[...], be2_ref[...])


def _tail(x, a0, a1, d0, d1, wo_t, bo, w1_t, b1, w2_t, b2, g1, be1, g2, be2):
    bm = 1000
    row = lambda w: pl.BlockSpec((bm, w), lambda i: (i, 0))
    full = lambda a, b: pl.BlockSpec((a, b), lambda i: (0, 0))
    return pl.pallas_call(
        _tail_body,
        grid=(N // bm,),
        in_specs=[
            row(D), row(D), row(D), row(H), row(H),
            full(D, D), full(1, D),
            full(D, FF), full(1, FF),
            full(FF, D), full(1, D),
            full(1, D), full(1, D), full(1, D), full(1, D),
        ],
        out_specs=pl.BlockSpec((bm, D), lambda i: (i, 0)),
        out_shape=jax.ShapeDtypeStruct((N, D), jnp.float32),
    )(x, a0, a1, d0, d1, wo_t, bo, w1_t, b1, w2_t, b2, g1, be1, g2, be2)


def kernel(x, edge_index, Wq, bq, Wk, bk, Wv, bv, Wo, bo, W1, b1, W2, b2,
           g1, beta1, g2, beta2):
    src = edge_index[0].astype(jnp.int32)
    dst = edge_index[1].astype(jnp.int32)
    ar = jnp.arange(D)
    perm = (ar % H) * OC + ar // H
    wqkv_t = jnp.concatenate(
        [Wq.T[:, perm], Wk.T[:, perm], Wv.T[:, perm]], axis=1)
    bqkv = jnp.concatenate([bq[perm], bk[perm], bv[perm]])[None, :]
    qtab, ktab, vtab = _qkv(x, wqkv_t, bqkv)
    acc_p = _edges(qtab, ktab, vtab, src, dst)
    a0 = acc_p[0, :N, :]
    a1 = acc_p[1, :N, :]
    den = acc_p[:, DEN0:DEN0 + N // 8, :].reshape(NC, N // 8, 8, L)
    den = den[..., :H].reshape(NC, N, H)
    d0 = den[0]
    d1 = den[1]
    return _tail(x, a0, a1, d0, d1, Wo.T[perm], bo[None, :], W1.T, b1[None, :],
                 W2.T, b2[None, :], g1[None, :], beta1[None, :],
                 g2[None, :], beta2[None, :])


# speedup vs baseline: 52.5961x; 1.0913x over previous
import functools

import jax
import jax.numpy as jnp
from jax import lax
from jax.experimental import pallas as pl
from jax.experimental.pallas import tpu as pltpu
from jax.experimental.pallas import tpu_sc as plsc

N = 10000
E = 320000
D = 128
H = 8
OC = D // H
FF = 512
SCALE = 1.0 / (OC ** 0.5)

NC, NS, L = 2, 16, 16
NW = NC * NS
EPW = E // NW
B = 16
NCHUNK = EPW // B
DEN0 = N
TROWS = 11264
SLAB = TROWS // NS


def _qkv_body(x_ref, w_ref, b_ref, q_ref, k_ref, v_ref):
    y = jnp.dot(x_ref[...], w_ref[...], preferred_element_type=jnp.float32)
    y = y + b_ref[...]
    q_ref[...] = y[:, :D] * SCALE
    k_ref[...] = y[:, D:2 * D]
    v_ref[...] = y[:, 2 * D:]


def _qkv(x, wqkv_t, bqkv):
    bm = 1000
    return pl.pallas_call(
        _qkv_body,
        grid=(N // bm,),
        in_specs=[
            pl.BlockSpec((bm, D), lambda i: (i, 0)),
            pl.BlockSpec((D, 3 * D), lambda i: (0, 0)),
            pl.BlockSpec((1, 3 * D), lambda i: (0, 0)),
        ],
        out_specs=[
            pl.BlockSpec((bm, D), lambda i: (i, 0)),
            pl.BlockSpec((bm, D), lambda i: (i, 0)),
            pl.BlockSpec((bm, D), lambda i: (i, 0)),
        ],
        out_shape=[
            jax.ShapeDtypeStruct((N, D), jnp.float32),
            jax.ShapeDtypeStruct((N, D), jnp.float32),
            jax.ShapeDtypeStruct((N, D), jnp.float32),
        ],
    )(x, wqkv_t, bqkv)


def _edge_body(qtab, ktab, vtab, srcv, dstv, out, sall, dall, vidx2, denid2,
               qr, kr, vr, zbuf, acc, sem0, sem1):
    c = lax.axis_index("c")
    s = lax.axis_index("s")
    wid = c * NS + s

    zero = jnp.zeros((L,), jnp.float32)

    def zrow(i, _):
        for j in range(D // L):
            zbuf[i, pl.ds(j * L, L)] = zero
        return 0

    lax.fori_loop(0, 32, zrow, 0)

    base = s * SLAB

    def zcopy(k, _):
        pltpu.sync_copy(zbuf, acc.at[pl.ds(base + k * 32, 32)])
        return 0

    lax.fori_loop(0, SLAB // 32, zcopy, 0)

    plsc.subcore_barrier()

    e0 = wid * EPW
    pltpu.sync_copy(srcv.at[pl.ds(e0, EPW)], sall)
    pltpu.sync_copy(dstv.at[pl.ds(e0, EPW)], dall)

    iota = lax.iota(jnp.int32, L)
    xor8 = iota ^ 8
    self_v = jnp.maximum(1.0 - iota.astype(jnp.float32), 0.0)
    sems = (sem0, sem1)

    def issue(g, b):
        dv = dall[pl.ds(g * B, L)]
        vidx2[b, pl.ds(0, L)] = dv
        denid2[b, pl.ds(0, L)] = (dv >> 3) + DEN0
        pltpu.async_copy(qtab.at[vidx2.at[b]], qr.at[b], sems[b])
        pltpu.async_copy(ktab.at[sall.at[pl.ds(g * B, B)]], kr.at[b], sems[b])
        pltpu.async_copy(vtab.at[sall.at[pl.ds(g * B, B)]], vr.at[b], sems[b])

    def drain(b):
        pltpu.make_async_copy(qtab.at[pl.ds(0, B)], qr.at[b], sems[b]).wait()
        pltpu.make_async_copy(ktab.at[pl.ds(0, B)], kr.at[b], sems[b]).wait()
        pltpu.make_async_copy(vtab.at[pl.ds(0, B)], vr.at[b], sems[b]).wait()

    def compute(b):
        dmv = vidx2[b, pl.ds(0, L)] & 7

        def edge4(t, _):
            for u in range(4):
                e = t * 4 + u
                a = qr[b, e, pl.ds(0, L)] * kr[b, e, pl.ds(0, L)]
                for j in range(1, D // L):
                    a = a + qr[b, e, pl.ds(j * L, L)] * kr[b, e, pl.ds(j * L, L)]
                a = a + jnp.take_along_axis(a, xor8, axis=0,
                                            mode='promise_in_bounds')
                ex = jnp.exp(a)
                for j in range(D // L):
                    vr[b, e, pl.ds(j * L, L)] = vr[b, e, pl.ds(j * L, L)] * ex
                bm = jnp.take_along_axis(
                    dmv, jnp.full((L,), e, jnp.int32), axis=0,
                    mode='promise_in_bounds')
                oh = jnp.take_along_axis(self_v, iota ^ bm, axis=0,
                                         mode='promise_in_bounds')
                for j in range(D // L):
                    m = jnp.take_along_axis(oh, jnp.full((L,), j, jnp.int32),
                                            axis=0,
                                            mode='promise_in_bounds')
                    kr[b, e, pl.ds(j * L, L)] = ex * m
            return 0

        lax.fori_loop(0, B // 4, edge4, 0)

    def scatter(b):
        pltpu.sync_copy(vr.at[b], acc.at[vidx2.at[b]], add=True)
        pltpu.sync_copy(kr.at[b], acc.at[denid2.at[b]], add=True)

    issue(0, 0)
    issue(1, 1)

    def pair(h, _):
        for b in range(2):
            g = h * 2 + b
            drain(b)
            compute(b)
            scatter(b)

            @pl.when(g + 2 < NCHUNK)
            def _():
                issue(g + 2, b)
        return 0

    lax.fori_loop(0, NCHUNK // 2, pair, 0)
    drain(0)
    compute(0)
    scatter(0)

    plsc.subcore_barrier()
    pltpu.sync_copy(acc.at[pl.ds(base, SLAB)], out.at[c, pl.ds(base, SLAB)])


def _edges(qtab, ktab, vtab, src, dst):
    mesh = plsc.VectorSubcoreMesh(core_axis_name="c", subcore_axis_name="s")
    f = pl.kernel(
        _edge_body,
        out_type=jax.ShapeDtypeStruct((NC, TROWS, D), jnp.float32),
        mesh=mesh,
        scratch_types=[
            pltpu.VMEM((EPW,), jnp.int32),
            pltpu.VMEM((EPW,), jnp.int32),
            pltpu.VMEM((2, B), jnp.int32),
            pltpu.VMEM((2, B), jnp.int32),
            pltpu.VMEM((2, B, D), jnp.float32),
            pltpu.VMEM((2, B, D), jnp.float32),
            pltpu.VMEM((2, B, D), jnp.float32),
            pltpu.VMEM((32, D), jnp.float32),
            pltpu.VMEM_SHARED((TROWS, D), jnp.float32),
            pltpu.SemaphoreType.DMA,
            pltpu.SemaphoreType.DMA,
        ],
    )
    return f(qtab, ktab, vtab, src, dst)


def _ln(x, g, b):
    mu = jnp.mean(x, axis=-1, keepdims=True)
    var = jnp.mean((x - mu) ** 2, axis=-1, keepdims=True)
    return (x - mu) / jnp.sqrt(var + 1e-5) * g + b


def _tail_body(x_ref, a0_ref, a1_ref, d0_ref, d1_ref, wo_ref, bo_ref,
               w1_ref, b1_ref, w2_ref, b2_ref, g1_ref, be1_ref, g2_ref,
               be2_ref, o_ref):
    den = d0_ref[...] + d1_ref[...]
    rep = (lax.broadcasted_iota(jnp.int32, (H, D), 1) % H
           == lax.broadcasted_iota(jnp.int32, (H, D), 0)).astype(jnp.float32)
    denr = jnp.dot(den, rep, preferred_element_type=jnp.float32)
    agg = (a0_ref[...] + a1_ref[...]) / (denr + 1e-16)
    att = jnp.dot(agg, wo_ref[...], preferred_element_type=jnp.float32) + bo_ref[...]
    x1 = x_ref[...] + att
    xn = _ln(x1, g1_ref[...], be1_ref[...])
    hh = jax.nn.gelu(jnp.dot(xn, w1_ref[...], preferred_element_type=jnp.float32)
                     + b1_ref[...])
    ff = jnp.dot(hh, w2_ref[...], preferred_element_type=jnp.float32) + b2_ref[...]
    o_ref[...] = _ln(xn + ff, g2_ref[...], be2_ref[...])


def _tail(x, a0, a1, d0, d1, wo_t, bo, w1_t, b1, w2_t, b2, g1, be1, g2, be2):
    bm = 1000
    row = lambda w: pl.BlockSpec((bm, w), lambda i: (i, 0))
    full = lambda a, b: pl.BlockSpec((a, b), lambda i: (0, 0))
    return pl.pallas_call(
        _tail_body,
        grid=(N // bm,),
        in_specs=[
            row(D), row(D), row(D), row(H), row(H),
            full(D, D), full(1, D),
            full(D, FF), full(1, FF),
            full(FF, D), full(1, D),
            full(1, D), full(1, D), full(1, D), full(1, D),
        ],
        out_specs=pl.BlockSpec((bm, D), lambda i: (i, 0)),
        out_shape=jax.ShapeDtypeStruct((N, D), jnp.float32),
    )(x, a0, a1, d0, d1, wo_t, bo, w1_t, b1, w2_t, b2, g1, be1, g2, be2)


def kernel(x, edge_index, Wq, bq, Wk, bk, Wv, bv, Wo, bo, W1, b1, W2, b2,
           g1, beta1, g2, beta2):
    src = edge_index[0].astype(jnp.int32)
    dst = edge_index[1].astype(jnp.int32)
    ar = jnp.arange(D)
    perm = (ar % H) * OC + ar // H
    wqkv_t = jnp.concatenate(
        [Wq.T[:, perm], Wk.T[:, perm], Wv.T[:, perm]], axis=1)
    bqkv = jnp.concatenate([bq[perm], bk[perm], bv[perm]])[None, :]
    qtab, ktab, vtab = _qkv(x, wqkv_t, bqkv)
    acc_p = _edges(qtab, ktab, vtab, src, dst)
    a0 = acc_p[0, :N, :]
    a1 = acc_p[1, :N, :]
    den = acc_p[:, DEN0:DEN0 + N // 8, :].reshape(NC, N // 8, 8, L)
    den = den[..., :H].reshape(NC, N, H)
    d0 = den[0]
    d1 = den[1]
    return _tail(x, a0, a1, d0, d1, Wo.T[perm], bo[None, :], W1.T, b1[None, :],
                 W2.T, b2[None, :], g1[None, :], beta1[None, :],
                 g2[None, :], beta2[None, :])
